# baseline (speedup 1.0000x reference)
import functools
import jax
import jax.numpy as jnp
from jax import lax
from jax.experimental import pallas as pl
from jax.experimental.pallas import tpu as pltpu
from jax.experimental.pallas import tpu_sc as plsc

BATCH = 4
SEQ = 8192
D_MODEL = 1024
NUM_CORES = 2
NUM_SUBCORES = 16
NUM_WORKERS = NUM_CORES * NUM_SUBCORES
ROWS_PER_WORKER = SEQ // NUM_WORKERS
TILE_ROWS = 8
NTILES = ROWS_PER_WORKER // TILE_ROWS
NGROUPS = 3
LANES = 16
VECS_PER_TILE = TILE_ROWS * D_MODEL // LANES
COLS = D_MODEL // LANES


def _sc_body(x_hbm, emb_hbm, out_hbm,
             e_v0, e_v1, e_v2,
             x_v0, x_v1, x_v2,
             se0, se1, se2,
             sx0, sx1, sx2,
             so0, so1, so2):
    wid = lax.axis_index("s") * NUM_CORES + lax.axis_index("c")
    base = wid * ROWS_PER_WORKER

    groups = (
        (e_v0, x_v0, se0, sx0, so0),
        (e_v1, x_v1, se1, sx1, so1),
        (e_v2, x_v2, se2, sx2, so2),
    )

    def issue_in(t, g):
        e_v, x_v, se, sx, _ = groups[g]
        row0 = base + t * TILE_ROWS
        pltpu.async_copy(emb_hbm.at[pl.ds(row0, TILE_ROWS)], e_v, se)
        pltpu.async_copy(x_hbm.at[:, pl.ds(row0, TILE_ROWS), :], x_v, sx)

    def wait_in(g):
        e_v, x_v, se, sx, _ = groups[g]
        pltpu.make_async_copy(emb_hbm.at[pl.ds(base, TILE_ROWS)], e_v, se).wait()
        pltpu.make_async_copy(x_hbm.at[:, pl.ds(base, TILE_ROWS), :], x_v, sx).wait()

    def wait_out(g):
        _, x_v, _, _, so = groups[g]
        pltpu.make_async_copy(x_v, out_hbm.at[:, pl.ds(base, TILE_ROWS), :], so).wait()

    def compute_and_out(t, g):
        e_v, x_v, _, _, so = groups[g]
        row0 = base + t * TILE_ROWS

        def vec_body(i, carry):
            r = i // COLS
            c0 = (i % COLS) * LANES
            e = e_v[r, pl.ds(c0, LANES)]
            for b in range(BATCH):
                x_v[b, r, pl.ds(c0, LANES)] = x_v[b, r, pl.ds(c0, LANES)] + e
            return carry

        lax.fori_loop(0, VECS_PER_TILE, vec_body, 0)
        pltpu.async_copy(x_v, out_hbm.at[:, pl.ds(row0, TILE_ROWS), :], so)

    def stage(t, g, do_wait_out=True, do_issue=True):
        if do_wait_out:
            wait_out((g + 1) % NGROUPS)
        if do_issue:
            issue_in(t + 1, (g + 1) % NGROUPS)
        wait_in(g)
        compute_and_out(t, g)

    issue_in(0, 0)
    stage(0, 0, do_wait_out=False)
    stage(1, 1, do_wait_out=False)

    def loop_body(k, carry):
        t0 = 2 + 3 * k
        for j in range(3):
            stage(t0 + j, (2 + j) % NGROUPS)
        return carry

    lax.fori_loop(0, (NTILES - 5) // 3, loop_body, 0)

    stage(NTILES - 3, (NTILES - 3) % NGROUPS)
    stage(NTILES - 2, (NTILES - 2) % NGROUPS)
    stage(NTILES - 1, (NTILES - 1) % NGROUPS, do_issue=False)
    wait_out((NTILES - 2) % NGROUPS)
    wait_out((NTILES - 1) % NGROUPS)


def kernel(x, emb_weight):
    mesh = plsc.VectorSubcoreMesh(core_axis_name="c", subcore_axis_name="s")
    run = functools.partial(
        pl.kernel,
        out_type=jax.ShapeDtypeStruct((BATCH, SEQ, D_MODEL), jnp.float32),
        mesh=mesh,
        scratch_types=[
            pltpu.VMEM((TILE_ROWS, D_MODEL), jnp.float32),
            pltpu.VMEM((TILE_ROWS, D_MODEL), jnp.float32),
            pltpu.VMEM((TILE_ROWS, D_MODEL), jnp.float32),
            pltpu.VMEM((BATCH, TILE_ROWS, D_MODEL), jnp.float32),
            pltpu.VMEM((BATCH, TILE_ROWS, D_MODEL), jnp.float32),
            pltpu.VMEM((BATCH, TILE_ROWS, D_MODEL), jnp.float32),
            pltpu.SemaphoreType.DMA,
            pltpu.SemaphoreType.DMA,
            pltpu.SemaphoreType.DMA,
            pltpu.SemaphoreType.DMA,
            pltpu.SemaphoreType.DMA,
            pltpu.SemaphoreType.DMA,
            pltpu.SemaphoreType.DMA,
            pltpu.SemaphoreType.DMA,
            pltpu.SemaphoreType.DMA,
        ],
    )(_sc_body)
    return run(x, emb_weight)


# speedup vs baseline: 1.0310x; 1.0310x over previous
import functools
import jax
import jax.numpy as jnp
from jax import lax
from jax.experimental import pallas as pl
from jax.experimental.pallas import tpu as pltpu
from jax.experimental.pallas import tpu_sc as plsc

BATCH = 4
SEQ = 8192
D_MODEL = 1024
NUM_CORES = 2
NUM_SUBCORES = 16
NUM_WORKERS = NUM_CORES * NUM_SUBCORES
ROWS_PER_WORKER = SEQ // NUM_WORKERS
TILE_ROWS = 4
NTILES = ROWS_PER_WORKER // TILE_ROWS
NGROUPS = 6
LANES = 16
VECS_PER_TILE = TILE_ROWS * D_MODEL // LANES
COLS = D_MODEL // LANES

_K = (NTILES - NGROUPS) // NGROUPS


def _sc_body(x_hbm, emb_hbm, out_hbm, *scr):
    e_bufs = scr[0:NGROUPS]
    x_bufs = scr[NGROUPS:2 * NGROUPS]
    se = scr[2 * NGROUPS:3 * NGROUPS]
    sx = scr[3 * NGROUPS:4 * NGROUPS]
    so = scr[4 * NGROUPS:5 * NGROUPS]

    wid = lax.axis_index("s") * NUM_CORES + lax.axis_index("c")
    base = wid * ROWS_PER_WORKER

    def issue_in(t, g):
        row0 = base + t * TILE_ROWS
        pltpu.async_copy(emb_hbm.at[pl.ds(row0, TILE_ROWS)], e_bufs[g], se[g])
        pltpu.async_copy(x_hbm.at[:, pl.ds(row0, TILE_ROWS), :], x_bufs[g], sx[g])

    def wait_in(g):
        pltpu.make_async_copy(
            emb_hbm.at[pl.ds(base, TILE_ROWS)], e_bufs[g], se[g]).wait()
        pltpu.make_async_copy(
            x_hbm.at[:, pl.ds(base, TILE_ROWS), :], x_bufs[g], sx[g]).wait()

    def wait_out(g):
        pltpu.make_async_copy(
            x_bufs[g], out_hbm.at[:, pl.ds(base, TILE_ROWS), :], so[g]).wait()

    def compute_and_out(t, g):
        e_v, x_v = e_bufs[g], x_bufs[g]
        row0 = base + t * TILE_ROWS

        def vec_body(i, carry):
            r = i // COLS
            c0 = (i % COLS) * LANES
            e = e_v[r, pl.ds(c0, LANES)]
            for b in range(BATCH):
                x_v[b, r, pl.ds(c0, LANES)] = x_v[b, r, pl.ds(c0, LANES)] + e
            return carry

        lax.fori_loop(0, VECS_PER_TILE, vec_body, 0)
        pltpu.async_copy(x_v, out_hbm.at[:, pl.ds(row0, TILE_ROWS), :], so[g])

    def stage(t, g, do_wait_out=True, do_issue=True):
        if do_wait_out:
            wait_out((g - 2) % NGROUPS)
        if do_issue:
            issue_in(t + NGROUPS - 2, (g - 2) % NGROUPS)
        wait_in(g)
        compute_and_out(t, g)

    for t in range(NGROUPS - 2):
        issue_in(t, t % NGROUPS)
    stage(0, 0, do_wait_out=False)
    stage(1, 1, do_wait_out=False)

    def loop_body(k, carry):
        t0 = 2 + NGROUPS * k
        for j in range(NGROUPS):
            stage(t0 + j, (2 + j) % NGROUPS)
        return carry

    lax.fori_loop(0, _K, loop_body, 0)

    for t in range(2 + NGROUPS * _K, NTILES):
        stage(t, t % NGROUPS, do_issue=(t + NGROUPS - 2 < NTILES))
    wait_out((NTILES - 2) % NGROUPS)
    wait_out((NTILES - 1) % NGROUPS)


def kernel(x, emb_weight):
    mesh = plsc.VectorSubcoreMesh(core_axis_name="c", subcore_axis_name="s")
    scratch = (
        [pltpu.VMEM((TILE_ROWS, D_MODEL), jnp.float32)] * NGROUPS
        + [pltpu.VMEM((BATCH, TILE_ROWS, D_MODEL), jnp.float32)] * NGROUPS
        + [pltpu.SemaphoreType.DMA] * (3 * NGROUPS)
    )
    run = functools.partial(
        pl.kernel,
        out_type=jax.ShapeDtypeStruct((BATCH, SEQ, D_MODEL), jnp.float32),
        mesh=mesh,
        scratch_types=scratch,
    )(_sc_body)
    return run(x, emb_weight)
